# Initial kernel scaffold; baseline (speedup 1.0000x reference)
import jax
import jax.numpy as jnp
from jax import lax
from jax.experimental import pallas as pl
from jax.experimental.pallas import tpu as pltpu
from jax.experimental.pallas import tpu_sc as plsc

NC, NS = 2, 16
NW = NC * NS
L = 16
S = 500
SPW = 16
SPAD = NW * SPW
D = 128
DG = D // L
C = 64


def _body(x_hbm, sizes_hbm, out_hbm, sizes_v, buf_v, tail_v, means_v):
    w = lax.axis_index("s") * NC + lax.axis_index("c")
    pltpu.sync_copy(sizes_hbm, sizes_v)
    s0 = w * SPW

    def psum_body(j, tot):
        return tot + jnp.sum(sizes_v[pl.ds(j * L, L)])

    start0 = lax.fori_loop(0, w, psum_body, jnp.int32(0))

    def seg_body(i, start):
        n = sizes_v[s0 + i]
        nfull = n // C
        rem = n - nfull * C

        def chunk_body(c, acc):
            pltpu.sync_copy(x_hbm.at[pl.ds(start + c * C, C)], buf_v)
            for r in range(C):
                acc = tuple(acc[f] + buf_v[r, pl.ds(f * L, L)]
                            for f in range(DG))
            return acc

        acc0 = tuple(jnp.zeros((L,), jnp.float32) for _ in range(DG))
        acc = lax.fori_loop(0, nfull, chunk_body, acc0)

        @pl.when(rem > 0)
        def _():
            pltpu.sync_copy(x_hbm.at[pl.ds(start + nfull * C, C)], tail_v)

        def row_body(r, acc):
            return tuple(acc[f] + tail_v[r, pl.ds(f * L, L)]
                         for f in range(DG))

        acc = lax.fori_loop(0, rem, row_body, acc)

        inv = 1.0 / jnp.maximum(n, 1).astype(jnp.float32)
        for f in range(DG):
            means_v[i, pl.ds(f * L, L)] = acc[f] * inv
        return start + n

    lax.fori_loop(0, SPW, seg_body, start0)

    tail_rows = S - (NW - 1) * SPW

    @pl.when(w < NW - 1)
    def _():
        pltpu.sync_copy(means_v, out_hbm.at[pl.ds(s0, SPW)])

    @pl.when(w == NW - 1)
    def _():
        pltpu.sync_copy(means_v.at[pl.ds(0, tail_rows)],
                        out_hbm.at[pl.ds(s0, tail_rows)])


_sc_call = pl.kernel(
    _body,
    out_type=jax.ShapeDtypeStruct((S, D), jnp.float32),
    mesh=plsc.VectorSubcoreMesh(core_axis_name="c", subcore_axis_name="s"),
    scratch_types=[
        pltpu.VMEM((SPAD,), jnp.int32),
        pltpu.VMEM((C, D), jnp.float32),
        pltpu.VMEM((C, D), jnp.float32),
        pltpu.VMEM((SPW, D), jnp.float32),
    ],
)


def kernel(x, feature_size_list):
    sizes = jnp.zeros((SPAD,), jnp.int32).at[:S].set(
        feature_size_list.astype(jnp.int32))
    return _sc_call(x, sizes)


# speedup vs baseline: 102.7521x; 102.7521x over previous
import jax
import jax.numpy as jnp
from jax import lax
from jax.experimental import pallas as pl
from jax.experimental.pallas import tpu as pltpu
from jax.experimental.pallas import tpu_sc as plsc

NC, NS = 2, 16
NW = NC * NS
L = 16
S = 500
SPW = 16
SPAD = NW * SPW
SALLOC = SPAD + L
D = 128
DG = D // L
C = 64


def _body(x_hbm, sizes_hbm, out_hbm, sizes_v, buf_v, means_v):
    w = lax.axis_index("s") * NC + lax.axis_index("c")
    pltpu.sync_copy(sizes_hbm, sizes_v)
    s0 = w * SPW

    def psum_body(j, tot):
        v = sizes_v[pl.ds(j * L, L)]
        for t in range(L):
            tot = tot + v[t]
        return tot

    start0 = lax.fori_loop(0, w, psum_body, jnp.int32(0))

    def seg_body(i, start):
        n = sizes_v[pl.ds(s0 + i, L)][0]
        end = start + n
        alo = (start // 8) * 8
        nch = (end - alo + C - 1) // C

        def chunk_body(c, acc):
            base = alo + c * C
            pltpu.sync_copy(x_hbm.at[pl.ds(base, C)], buf_v)
            lo = jnp.maximum(start - base, 0)
            hi = jnp.minimum(end - base, C)

            def row_body(r, acc):
                return tuple(acc[f] + buf_v[r, pl.ds(f * L, L)]
                             for f in range(DG))

            return lax.fori_loop(lo, hi, row_body, acc)

        acc0 = tuple(jnp.zeros((L,), jnp.float32) for _ in range(DG))
        acc = lax.fori_loop(0, nch, chunk_body, acc0)

        n_vec = jnp.full((L,), jnp.maximum(n, 1),
                         dtype=jnp.int32).astype(jnp.float32)
        for f in range(DG):
            means_v[i, pl.ds(f * L, L)] = acc[f] / n_vec
        return end

    lax.fori_loop(0, SPW, seg_body, start0)
    pltpu.sync_copy(means_v, out_hbm.at[pl.ds(s0, SPW)])


_sc_call = pl.kernel(
    _body,
    out_type=jax.ShapeDtypeStruct((SPAD, D), jnp.float32),
    mesh=plsc.VectorSubcoreMesh(core_axis_name="c", subcore_axis_name="s"),
    scratch_types=[
        pltpu.VMEM((SALLOC,), jnp.int32),
        pltpu.VMEM((C, D), jnp.float32),
        pltpu.VMEM((SPW, D), jnp.float32),
    ],
)


def kernel(x, feature_size_list):
    sizes = jnp.zeros((SALLOC,), jnp.int32).at[:S].set(
        feature_size_list.astype(jnp.int32))
    return _sc_call(x, sizes)[:S]


# speedup vs baseline: 164.4593x; 1.6005x over previous
import jax
import jax.numpy as jnp
from jax import lax
from jax.experimental import pallas as pl
from jax.experimental.pallas import tpu as pltpu
from jax.experimental.pallas import tpu_sc as plsc

NC, NS = 2, 16
NW = NC * NS
L = 16
S = 500
SPW = 16
SPAD = NW * SPW
SALLOC = SPAD + L
D = 128
DG = D // L
C = 64


def _body(x_hbm, sizes_hbm, out_hbm, sizes_v, buf_v, means_v, sem):
    w = lax.axis_index("s") * NC + lax.axis_index("c")
    pltpu.sync_copy(sizes_hbm, sizes_v)
    s0 = w * SPW

    def psum_body(j, tot):
        v = sizes_v[pl.ds(j * L, L)]
        for t in range(L):
            tot = tot + v[t]
        return tot

    start0 = lax.fori_loop(0, w, psum_body, jnp.int32(0))

    def issue(p, base):
        pltpu.async_copy(x_hbm.at[pl.ds(base, C)], buf_v.at[p], sem.at[p])

    def wait(p):
        pltpu.make_async_copy(x_hbm.at[pl.ds(0, C)], buf_v.at[p],
                              sem.at[p]).wait()

    issue(jnp.int32(0), (start0 // 8) * 8)

    def seg_body(i, carry):
        start, pbase = carry
        n = sizes_v[pl.ds(s0 + i, L)][0]
        n_next = sizes_v[pl.ds(s0 + i + 1, L)][0]
        end = start + n
        alo = (start // 8) * 8
        nch = (end - alo + C - 1) // C

        def process():
            def chunk_body(c, acc):
                p = (pbase + c) & 1
                base = alo + c * C

                @pl.when(c + 1 < nch)
                def _():
                    issue((pbase + c + 1) & 1, base + C)

                @pl.when(jnp.logical_and(
                    c + 1 == nch,
                    jnp.logical_and(i + 1 < SPW, n_next > 0)))
                def _():
                    issue((pbase + nch) & 1, (end // 8) * 8)

                wait(p)
                lo = jnp.maximum(start - base, 0)
                hi = jnp.minimum(end - base, C)

                def row_body(r, a):
                    return tuple(a[f] + buf_v[p, r, pl.ds(f * L, L)]
                                 for f in range(DG))

                return lax.fori_loop(lo, hi, row_body, acc)

            acc0 = tuple(jnp.zeros((L,), jnp.float32) for _ in range(DG))
            acc = lax.fori_loop(0, nch, chunk_body, acc0)

            n_vec = jnp.full((L,), jnp.maximum(n, 1),
                             dtype=jnp.int32).astype(jnp.float32)
            for f in range(DG):
                means_v[i, pl.ds(f * L, L)] = acc[f] / n_vec

        @pl.when(n > 0)
        def _():
            process()

        return end, (pbase + nch) & 1

    lax.fori_loop(0, SPW, seg_body, (start0, jnp.int32(0)))
    pltpu.sync_copy(means_v, out_hbm.at[pl.ds(s0, SPW)])


_sc_call = pl.kernel(
    _body,
    out_type=jax.ShapeDtypeStruct((SPAD, D), jnp.float32),
    mesh=plsc.VectorSubcoreMesh(core_axis_name="c", subcore_axis_name="s"),
    scratch_types=[
        pltpu.VMEM((SALLOC,), jnp.int32),
        pltpu.VMEM((2, C, D), jnp.float32),
        pltpu.VMEM((SPW, D), jnp.float32),
        pltpu.SemaphoreType.DMA((2,)),
    ],
)


def kernel(x, feature_size_list):
    sizes = jnp.zeros((SALLOC,), jnp.int32).at[:S].set(
        feature_size_list.astype(jnp.int32))
    return _sc_call(x, sizes)[:S]


# speedup vs baseline: 169.2125x; 1.0289x over previous
import jax
import jax.numpy as jnp
from jax import lax
from jax.experimental import pallas as pl
from jax.experimental.pallas import tpu as pltpu
from jax.experimental.pallas import tpu_sc as plsc

NC, NS = 2, 16
NW = NC * NS
L = 16
S = 500
SPW = 16
SPAD = NW * SPW
SALLOC = SPAD + L
D = 128
DG = D // L
C = 128


def _body(x_hbm, sizes_hbm, out_hbm, sizes_v, buf_v, means_v, sem):
    w = lax.axis_index("s") * NC + lax.axis_index("c")
    pltpu.sync_copy(sizes_hbm, sizes_v)
    s0 = w * SPW

    def psum_body(j, tot):
        v = sizes_v[pl.ds(j * L, L)]
        for t in range(L):
            tot = tot + v[t]
        return tot

    start0 = lax.fori_loop(0, w, psum_body, jnp.int32(0))

    def issue(p, base):
        pltpu.async_copy(x_hbm.at[pl.ds(base, C)], buf_v.at[p], sem.at[p])

    def wait(p):
        pltpu.make_async_copy(x_hbm.at[pl.ds(0, C)], buf_v.at[p],
                              sem.at[p]).wait()

    issue(jnp.int32(0), (start0 // 8) * 8)

    def seg_body(i, carry):
        start, pbase = carry
        n = sizes_v[pl.ds(s0 + i, L)][0]
        n_next = sizes_v[pl.ds(s0 + i + 1, L)][0]
        end = start + n
        alo = (start // 8) * 8
        nch = (end - alo + C - 1) // C

        def process():
            def chunk_body(c, acc):
                p = (pbase + c) & 1
                base = alo + c * C

                @pl.when(c + 1 < nch)
                def _():
                    issue((pbase + c + 1) & 1, base + C)

                @pl.when(jnp.logical_and(
                    c + 1 == nch,
                    jnp.logical_and(i + 1 < SPW, n_next > 0)))
                def _():
                    issue((pbase + nch) & 1, (end // 8) * 8)

                wait(p)
                lo = jnp.maximum(start - base, 0)
                hi = jnp.minimum(end - base, C)

                def row_body(r, a):
                    return tuple(a[f] + buf_v[p, r, pl.ds(f * L, L)]
                                 for f in range(DG))

                return lax.fori_loop(lo, hi, row_body, acc)

            acc0 = tuple(jnp.zeros((L,), jnp.float32) for _ in range(DG))
            acc = lax.fori_loop(0, nch, chunk_body, acc0)

            n_vec = jnp.full((L,), jnp.maximum(n, 1),
                             dtype=jnp.int32).astype(jnp.float32)
            for f in range(DG):
                means_v[i, pl.ds(f * L, L)] = acc[f] / n_vec

        @pl.when(n > 0)
        def _():
            process()

        return end, (pbase + nch) & 1

    lax.fori_loop(0, SPW, seg_body, (start0, jnp.int32(0)))
    pltpu.sync_copy(means_v, out_hbm.at[pl.ds(s0, SPW)])


_sc_call = pl.kernel(
    _body,
    out_type=jax.ShapeDtypeStruct((SPAD, D), jnp.float32),
    mesh=plsc.VectorSubcoreMesh(core_axis_name="c", subcore_axis_name="s"),
    scratch_types=[
        pltpu.VMEM((SALLOC,), jnp.int32),
        pltpu.VMEM((2, C, D), jnp.float32),
        pltpu.VMEM((SPW, D), jnp.float32),
        pltpu.SemaphoreType.DMA((2,)),
    ],
)


def kernel(x, feature_size_list):
    sizes = jnp.zeros((SALLOC,), jnp.int32).at[:S].set(
        feature_size_list.astype(jnp.int32))
    return _sc_call(x, sizes)[:S]


# speedup vs baseline: 172.3627x; 1.0186x over previous
import jax
import jax.numpy as jnp
from jax import lax
from jax.experimental import pallas as pl
from jax.experimental.pallas import tpu as pltpu
from jax.experimental.pallas import tpu_sc as plsc

NC, NS = 2, 16
NW = NC * NS
L = 16
S = 500
SPW = 16
SPAD = NW * SPW
SALLOC = SPAD + L
D = 128
DG = D // L
C = 128


def _body(x_hbm, sizes_hbm, out_hbm, sizes_v, buf_v, means_v, sem, osem):
    w = lax.axis_index("s") * NC + lax.axis_index("c")
    pltpu.sync_copy(sizes_hbm, sizes_v)

    def t_body(j, tot):
        v = sizes_v[pl.ds(j * L, L)]
        for t in range(L):
            tot = tot + v[t]
        return tot

    total = lax.fori_loop(0, SPAD // L, t_body, jnp.int32(0))

    def walk_body(j, carry):
        cum, s_begin, s_end, row_begin, row_end, found = carry
        v = sizes_v[pl.ds(j * L, L)]
        for t in range(L):
            size = v[t]
            s = j * L + t
            m = (2 * cum + size) * 16
            mine = jnp.logical_and(
                jnp.logical_and(m >= w * total, m < (w + 1) * total),
                size > 0).astype(jnp.int32)
            first = mine * (1 - found)
            s_begin = first * s + (1 - first) * s_begin
            row_begin = first * cum + (1 - first) * row_begin
            s_end = mine * (s + 1) + (1 - mine) * s_end
            row_end = mine * (cum + size) + (1 - mine) * row_end
            found = jnp.maximum(found, mine)
            cum = cum + size
        return cum, s_begin, s_end, row_begin, row_end, found

    z = jnp.int32(0)
    _, s_begin, s_end, row_begin, _, found = lax.fori_loop(
        0, SPAD // L, walk_body, (z, z, z, z, z, z))
    s_count = (s_end - s_begin) * found

    def issue(p, base):
        pltpu.async_copy(x_hbm.at[pl.ds(base, C)], buf_v.at[p], sem.at[p])

    def wait(p):
        pltpu.make_async_copy(x_hbm.at[pl.ds(0, C)], buf_v.at[p],
                              sem.at[p]).wait()

    @pl.when(s_count > 0)
    def _process_all():
        issue(jnp.int32(0), (row_begin // 8) * 8)

        def seg_body(i, carry):
            start, pbase = carry
            n = sizes_v[pl.ds(s_begin + i, L)][0]
            end = start + n
            alo = (start // 8) * 8
            nch = (end - alo + C - 1) // C

            def chunk_body(c, acc):
                p = (pbase + c) & 1
                base = alo + c * C

                @pl.when(c + 1 < nch)
                def _():
                    issue((pbase + c + 1) & 1, base + C)

                @pl.when(jnp.logical_and(c + 1 == nch, i + 1 < s_count))
                def _():
                    issue((pbase + nch) & 1, (end // 8) * 8)

                wait(p)
                lo = jnp.maximum(start - base, 0)
                hi = jnp.minimum(end - base, C)

                def row_body(r, a):
                    return tuple(a[f] + buf_v[p, r, pl.ds(f * L, L)]
                                 for f in range(DG))

                return lax.fori_loop(lo, hi, row_body, acc)

            acc0 = tuple(jnp.zeros((L,), jnp.float32) for _ in range(DG))
            acc = lax.fori_loop(0, nch, chunk_body, acc0)

            n_vec = jnp.full((L,), jnp.maximum(n, 1),
                             dtype=jnp.int32).astype(jnp.float32)
            for f in range(DG):
                means_v[i, pl.ds(f * L, L)] = acc[f] / n_vec
            return end, (pbase + nch) & 1

        lax.fori_loop(0, s_count, seg_body, (row_begin, jnp.int32(0)))

        ngroups = (s_count + L - 1) // L

        def pad_body(k, o):
            for f in range(DG):
                means_v[k, pl.ds(f * L, L)] = \
                    means_v[s_count - 1, pl.ds(f * L, L)]
            return o

        lax.fori_loop(s_count, ngroups * L, pad_body, z)

        def out_body(g, o):
            idx = jnp.minimum(s_begin + g * L + lax.iota(jnp.int32, 16),
                              s_end - 1)
            src = means_v.at[pl.ds(g * L, L)]
            pltpu.async_copy(src, out_hbm.at[idx], osem)
            pltpu.make_async_copy(src, out_hbm.at[idx], osem).wait()
            return o

        lax.fori_loop(0, ngroups, out_body, z)


_sc_call = pl.kernel(
    _body,
    out_type=jax.ShapeDtypeStruct((SPAD, D), jnp.float32),
    mesh=plsc.VectorSubcoreMesh(core_axis_name="c", subcore_axis_name="s"),
    scratch_types=[
        pltpu.VMEM((SALLOC,), jnp.int32),
        pltpu.VMEM((2, C, D), jnp.float32),
        pltpu.VMEM((SPAD, D), jnp.float32),
        pltpu.SemaphoreType.DMA((2,)),
        pltpu.SemaphoreType.DMA,
    ],
)


def kernel(x, feature_size_list):
    sizes = jnp.zeros((SALLOC,), jnp.int32).at[:S].set(
        feature_size_list.astype(jnp.int32))
    return _sc_call(x, sizes)[:S]


# speedup vs baseline: 215.5658x; 1.2507x over previous
import jax
import jax.numpy as jnp
from jax import lax
from jax.experimental import pallas as pl
from jax.experimental.pallas import tpu as pltpu
from jax.experimental.pallas import tpu_sc as plsc

NC, NS = 2, 16
NW = NC * NS
L = 16
S = 500
SPW = 16
SPAD = NW * SPW
SALLOC = SPAD + L
D = 128
DG = D // L
C = 96


def _body(x_hbm, sizes_hbm, out_hbm, sizes_v, buf_v, means_v, sem, osem):
    w = lax.axis_index("s") * NC + lax.axis_index("c")
    pltpu.sync_copy(sizes_hbm, sizes_v)

    def t_body(j, tot):
        v = sizes_v[pl.ds(j * L, L)]
        for t in range(L):
            tot = tot + v[t]
        return tot

    total = lax.fori_loop(0, SPAD // L, t_body, jnp.int32(0))

    def walk_body(j, carry):
        cum, s_begin, s_end, row_begin, row_end, found = carry
        v = sizes_v[pl.ds(j * L, L)]
        for t in range(L):
            size = v[t]
            s = j * L + t
            m = (2 * cum + size) * 16
            mine = jnp.logical_and(
                jnp.logical_and(m >= w * total, m < (w + 1) * total),
                size > 0).astype(jnp.int32)
            first = mine * (1 - found)
            s_begin = first * s + (1 - first) * s_begin
            row_begin = first * cum + (1 - first) * row_begin
            s_end = mine * (s + 1) + (1 - mine) * s_end
            row_end = mine * (cum + size) + (1 - mine) * row_end
            found = jnp.maximum(found, mine)
            cum = cum + size
        return cum, s_begin, s_end, row_begin, row_end, found

    z = jnp.int32(0)
    _, s_begin, s_end, row_begin, row_end, found = lax.fori_loop(
        0, SPAD // L, walk_body, (z, z, z, z, z, z))
    s_count = (s_end - s_begin) * found

    def issue(p, base):
        pltpu.async_copy(x_hbm.at[pl.ds(base, C)], buf_v.at[p], sem.at[p])

    def wait(p):
        pltpu.make_async_copy(x_hbm.at[pl.ds(0, C)], buf_v.at[p],
                              sem.at[p]).wait()

    @pl.when(s_count > 0)
    def _process_all():
        alo = (row_begin // 8) * 8
        nch_tot = (row_end - alo + C - 1) // C
        issue(jnp.int32(0), alo)

        @pl.when(nch_tot > 1)
        def _():
            issue(jnp.int32(1), alo + C)

        def seg_body(i, carry):
            start, loaded = carry
            n = sizes_v[pl.ds(s_begin + i, L)][0]
            end = start + n
            c_lo = (start - alo) // C
            c_hi = (end - 1 - alo) // C

            def chunk_body(c, carry):
                acc, loaded = carry
                base = alo + c * C

                @pl.when(c > loaded)
                def _():
                    wait(c & 3)

                    @pl.when(c + 2 < nch_tot)
                    def _():
                        issue((c + 2) & 3, alo + (c + 2) * C)

                lo = jnp.maximum(start - base, 0)
                hi = jnp.minimum(end - base, C)
                p = c & 3

                def row_body(r, a):
                    return tuple(a[f] + buf_v[p, r, pl.ds(f * L, L)]
                                 for f in range(DG))

                return (lax.fori_loop(lo, hi, row_body, acc),
                        jnp.maximum(loaded, c))

            acc0 = tuple(jnp.zeros((L,), jnp.float32) for _ in range(DG))
            acc, loaded = lax.fori_loop(c_lo, c_hi + 1, chunk_body,
                                        (acc0, loaded))

            n_vec = jnp.full((L,), jnp.maximum(n, 1),
                             dtype=jnp.int32).astype(jnp.float32)
            for f in range(DG):
                means_v[i, pl.ds(f * L, L)] = acc[f] / n_vec
            return end, loaded

        lax.fori_loop(0, s_count, seg_body, (row_begin, jnp.int32(-1)))

        ngroups = (s_count + L - 1) // L

        def pad_body(k, o):
            for f in range(DG):
                means_v[k, pl.ds(f * L, L)] = \
                    means_v[s_count - 1, pl.ds(f * L, L)]
            return o

        lax.fori_loop(s_count, ngroups * L, pad_body, z)

        def out_body(g, o):
            idx = jnp.minimum(s_begin + g * L + lax.iota(jnp.int32, 16),
                              s_end - 1)
            src = means_v.at[pl.ds(g * L, L)]
            pltpu.async_copy(src, out_hbm.at[idx], osem)
            pltpu.make_async_copy(src, out_hbm.at[idx], osem).wait()
            return o

        lax.fori_loop(0, ngroups, out_body, z)


_sc_call = pl.kernel(
    _body,
    out_type=jax.ShapeDtypeStruct((SPAD, D), jnp.float32),
    mesh=plsc.VectorSubcoreMesh(core_axis_name="c", subcore_axis_name="s"),
    scratch_types=[
        pltpu.VMEM((SALLOC,), jnp.int32),
        pltpu.VMEM((4, C, D), jnp.float32),
        pltpu.VMEM((SPAD, D), jnp.float32),
        pltpu.SemaphoreType.DMA((4,)),
        pltpu.SemaphoreType.DMA,
    ],
)


def kernel(x, feature_size_list):
    sizes = jnp.zeros((SALLOC,), jnp.int32).at[:S].set(
        feature_size_list.astype(jnp.int32))
    return _sc_call(x, sizes)[:S]


# speedup vs baseline: 237.2943x; 1.1008x over previous
import jax
import jax.numpy as jnp
from jax import lax
from jax.experimental import pallas as pl
from jax.experimental.pallas import tpu as pltpu
from jax.experimental.pallas import tpu_sc as plsc

NC, NS = 2, 16
NW = NC * NS
L = 16
S = 500
SPW = 16
SPAD = NW * SPW
SALLOC = SPAD + L
D = 128
DG = D // L
C = 128
NB = 3


def _body(x_hbm, sizes_hbm, out_hbm, sizes_v, buf_v, means_v, sem, osem):
    w = lax.axis_index("s") * NC + lax.axis_index("c")
    pltpu.sync_copy(sizes_hbm, sizes_v)

    def t_body(j, tot):
        v = sizes_v[pl.ds(j * L, L)]
        for t in range(L):
            tot = tot + v[t]
        return tot

    total = lax.fori_loop(0, SPAD // L, t_body, jnp.int32(0))

    def walk_body(j, carry):
        cum, s_begin, s_end, row_begin, row_end, found = carry
        v = sizes_v[pl.ds(j * L, L)]
        for t in range(L):
            size = v[t]
            s = j * L + t
            m = (2 * cum + size) * 16
            mine = jnp.logical_and(
                jnp.logical_and(m >= w * total, m < (w + 1) * total),
                size > 0)
            first = jnp.logical_and(mine, found == 0)
            s_begin = jnp.where(first, s, s_begin)
            row_begin = jnp.where(first, cum, row_begin)
            s_end = jnp.where(mine, s + 1, s_end)
            row_end = jnp.where(mine, cum + size, row_end)
            found = jnp.where(mine, jnp.int32(1), found)
            cum = cum + size
        return cum, s_begin, s_end, row_begin, row_end, found

    z = jnp.int32(0)
    _, s_begin, s_end, row_begin, row_end, found = lax.fori_loop(
        0, SPAD // L, walk_body, (z, z, z, z, z, z))
    s_count = (s_end - s_begin) * found

    def issue(p, base):
        pltpu.async_copy(x_hbm.at[pl.ds(base, C)], buf_v.at[p], sem.at[p])

    def wait(p):
        pltpu.make_async_copy(x_hbm.at[pl.ds(0, C)], buf_v.at[p],
                              sem.at[p]).wait()

    @pl.when(s_count > 0)
    def _process_all():
        alo = (row_begin // 8) * 8
        nch_tot = (row_end - alo + C - 1) // C
        issue(jnp.int32(0), alo)

        @pl.when(nch_tot > 1)
        def _():
            issue(jnp.int32(1), alo + C)

        def seg_body(i, carry):
            start, loaded = carry
            n = sizes_v[pl.ds(s_begin + i, L)][0]
            end = start + n
            c_lo = (start - alo) // C
            c_hi = (end - 1 - alo) // C

            def chunk_body(c, carry):
                acc, loaded = carry
                base = alo + c * C
                p = lax.rem(c, jnp.int32(NB))

                @pl.when(c > loaded)
                def _():
                    wait(p)

                    @pl.when(c + 2 < nch_tot)
                    def _():
                        issue(lax.rem(c + 2, jnp.int32(NB)),
                              alo + (c + 2) * C)

                lo = jnp.maximum(start - base, 0)
                hi = jnp.minimum(end - base, C)

                def row_body(r, a):
                    return tuple(a[f] + buf_v[p, r, pl.ds(f * L, L)]
                                 for f in range(DG))

                return (lax.fori_loop(lo, hi, row_body, acc),
                        jnp.maximum(loaded, c))

            acc0 = tuple(jnp.zeros((L,), jnp.float32) for _ in range(DG))
            acc, loaded = lax.fori_loop(c_lo, c_hi + 1, chunk_body,
                                        (acc0, loaded))

            n_vec = jnp.full((L,), jnp.maximum(n, 1),
                             dtype=jnp.int32).astype(jnp.float32)
            for f in range(DG):
                means_v[i, pl.ds(f * L, L)] = acc[f] / n_vec
            return end, loaded

        lax.fori_loop(0, s_count, seg_body, (row_begin, jnp.int32(-1)))

        ngroups = (s_count + L - 1) // L

        def pad_body(k, o):
            for f in range(DG):
                means_v[k, pl.ds(f * L, L)] = \
                    means_v[s_count - 1, pl.ds(f * L, L)]
            return o

        lax.fori_loop(s_count, ngroups * L, pad_body, z)

        def out_body(g, o):
            idx = jnp.minimum(s_begin + g * L + lax.iota(jnp.int32, 16),
                              s_end - 1)
            src = means_v.at[pl.ds(g * L, L)]
            pltpu.async_copy(src, out_hbm.at[idx], osem)
            pltpu.make_async_copy(src, out_hbm.at[idx], osem).wait()
            return o

        lax.fori_loop(0, ngroups, out_body, z)


_sc_call = pl.kernel(
    _body,
    out_type=jax.ShapeDtypeStruct((SPAD, D), jnp.float32),
    mesh=plsc.VectorSubcoreMesh(core_axis_name="c", subcore_axis_name="s"),
    scratch_types=[
        pltpu.VMEM((SALLOC,), jnp.int32),
        pltpu.VMEM((NB, C, D), jnp.float32),
        pltpu.VMEM((SPAD, D), jnp.float32),
        pltpu.SemaphoreType.DMA((NB,)),
        pltpu.SemaphoreType.DMA,
    ],
)


def kernel(x, feature_size_list):
    sizes = jnp.zeros((SALLOC,), jnp.int32).at[:S].set(
        feature_size_list.astype(jnp.int32))
    return _sc_call(x, sizes)[:S]
